# baseline (speedup 1.0000x reference)
import jax
import jax.numpy as jnp
from jax import lax
from jax.experimental import pallas as pl
from jax.experimental.pallas import tpu as pltpu
from jax.experimental.pallas import tpu_sc as plsc

NUM_WORKERS = 32
CHUNK_ROWS = 4
LANES = 16


def _build(B, S, D, n_table):
    assert B % NUM_WORKERS == 0
    rows_per_w = B // NUM_WORKERS
    assert rows_per_w % (2 * CHUNK_ROWS) == 0
    n_chunks = rows_per_w // CHUNK_ROWS
    CS = CHUNK_ROWS * S
    assert D % LANES == 0
    d_regs = D // LANES
    n_full_groups = S // LANES
    tail = S - n_full_groups * LANES
    slices = []
    off = 0
    while off < CS:
        n = min(128, CS - off)
        slices.append((off, n))
        off += n

    mesh = plsc.VectorSubcoreMesh(core_axis_name="c", subcore_axis_name="s")

    def lane_total(v):
        for s in (8, 4, 2, 1):
            perm = lax.iota(jnp.int32, LANES) ^ jnp.int32(s)
            v = v + jnp.take_along_axis(v, perm, axis=0)
        return v

    def body(x_hbm, m_hbm, table_hbm, out_hbm,
             xv0, mv0, rows0, cnt0, xv1, mv1, rows1, cnt1, z0v, outb,
             gsem0, gsem1):
        wid = lax.axis_index("s") * 2 + lax.axis_index("c")
        bufs = ((xv0, mv0, rows0, cnt0, gsem0), (xv1, mv1, rows1, cnt1, gsem1))

        pltpu.sync_copy(table_hbm.at[pl.ds(0, 8)], z0v)

        def load_and_select(c, buf):
            xv, mv, _, cntb, _ = buf
            base = (wid * rows_per_w + c * CHUNK_ROWS) * S
            pltpu.sync_copy(x_hbm.at[pl.ds(base, CS)], xv)
            pltpu.sync_copy(m_hbm.at[pl.ds(base, CS)], mv.at[pl.ds(0, CS)])
            zi = jnp.zeros((LANES,), jnp.int32)
            zf = jnp.zeros((LANES,), jnp.float32)
            for r in range(CHUNK_ROWS):
                rb = r * S
                cntv = zf
                for g in range(n_full_groups):
                    o = rb + g * LANES
                    m16 = mv[pl.ds(o, LANES)]
                    x16 = xv[pl.ds(o, LANES)]
                    xv[pl.ds(o, LANES)] = jnp.where(m16 > 0, x16, zi)
                    cntv = cntv + m16
                o = rb + S - LANES
                m16 = mv[pl.ds(o, LANES)]
                x16 = xv[pl.ds(o, LANES)]
                xv[pl.ds(o, LANES)] = jnp.where(m16 > 0, x16, zi)
                mtail = jnp.where(
                    lax.iota(jnp.int32, LANES) >= (LANES - tail),
                    m16,
                    jnp.float32(0.0),
                )
                cntv = cntv + mtail
                cntb[r] = lane_total(cntv)

        def gather_copies(buf):
            xv, _, rows_v, _, gsem = buf
            for off, n in slices:
                yield pltpu.make_async_copy(
                    table_hbm.at[xv.at[pl.ds(off, n)]],
                    rows_v.at[pl.ds(off, n)],
                    gsem,
                )

        def fire(buf):
            for cp in gather_copies(buf):
                cp.start()

        def wait(buf):
            for cp in gather_copies(buf):
                cp.wait()

        def process(c, buf):
            _, _, rows_v, cntb, _ = buf
            row0 = wid * rows_per_w + c * CHUNK_ROWS
            z0 = [z0v[0, pl.ds(d * LANES, LANES)] for d in range(d_regs)]
            for r in range(CHUNK_ROWS):
                rb = r * S

                def gbody(g, accs):
                    base = rb + g * LANES
                    out = list(accs)
                    for j in range(LANES):
                        for d in range(d_regs):
                            out[d] = out[d] + rows_v[base + j, pl.ds(d * LANES, LANES)]
                    return tuple(out)

                z = jnp.zeros((LANES,), jnp.float32)
                accs = lax.fori_loop(0, n_full_groups, gbody, (z,) * d_regs)
                accs = list(accs)
                tbase = rb + n_full_groups * LANES
                for j in range(tail):
                    for d in range(d_regs):
                        accs[d] = accs[d] + rows_v[tbase + j, pl.ds(d * LANES, LANES)]
                cntv = cntb[r]
                extra = jnp.float32(float(S)) - cntv
                inv = jnp.float32(1.0) / (cntv + jnp.float32(1e-9))
                for d in range(d_regs):
                    outb[r, pl.ds(d * LANES, LANES)] = (
                        accs[d] - extra * z0[d]
                    ) * inv

            pltpu.sync_copy(outb, out_hbm.at[pl.ds(row0, CHUNK_ROWS)])

        load_and_select(0, bufs[0])
        fire(bufs[0])

        def pair_body(i, carry):
            c0 = 2 * i
            load_and_select(c0 + 1, bufs[1])
            fire(bufs[1])
            wait(bufs[0])
            process(c0, bufs[0])

            @pl.when(c0 + 2 < n_chunks)
            def _():
                load_and_select(c0 + 2, bufs[0])
                fire(bufs[0])

            wait(bufs[1])
            process(c0 + 1, bufs[1])
            return carry

        lax.fori_loop(0, n_chunks // 2, pair_body, 0)

    return pl.kernel(
        body,
        out_type=jax.ShapeDtypeStruct((B, D), jnp.float32),
        mesh=mesh,
        compiler_params=pltpu.CompilerParams(use_tc_tiling_on_sc=False),
        scratch_types=[
            pltpu.VMEM((CS,), jnp.int32),
            pltpu.VMEM((CS + LANES,), jnp.float32),
            pltpu.VMEM((CS, D), jnp.float32),
            pltpu.VMEM((CHUNK_ROWS, LANES), jnp.float32),
            pltpu.VMEM((CS,), jnp.int32),
            pltpu.VMEM((CS + LANES,), jnp.float32),
            pltpu.VMEM((CS, D), jnp.float32),
            pltpu.VMEM((CHUNK_ROWS, LANES), jnp.float32),
            pltpu.VMEM((8, D), jnp.float32),
            pltpu.VMEM((CHUNK_ROWS, D), jnp.float32),
            pltpu.SemaphoreType.DMA,
            pltpu.SemaphoreType.DMA,
        ],
    )


@jax.jit
def kernel(x, mask, table):
    B, S = x.shape
    n_table, D = table.shape
    xf = x.reshape(-1).astype(jnp.int32)
    mf = mask.reshape(-1).astype(jnp.float32)
    return _build(B, S, D, n_table)(xf, mf, table)


# speedup vs baseline: 17.8183x; 17.8183x over previous
import jax
import jax.numpy as jnp
from jax import lax
from jax.experimental import pallas as pl
from jax.experimental.pallas import tpu as pltpu
from jax.experimental.pallas import tpu_sc as plsc

NUM_WORKERS = 32
CHUNK_ROWS = 4
LANES = 16
UNROLL = 8


def _build(B, S, D, n_table):
    assert B % NUM_WORKERS == 0
    rows_per_w = B // NUM_WORKERS
    assert rows_per_w % (2 * CHUNK_ROWS) == 0
    n_chunks = rows_per_w // CHUNK_ROWS
    CS = CHUNK_ROWS * S
    assert D % LANES == 0
    d_regs = D // LANES
    n_full_groups = S // LANES
    tail = S - n_full_groups * LANES
    max_comp = CS + CHUNK_ROWS * (UNROLL - 1)
    n_slices = -(-max_comp // 128)
    rows_cap = n_slices * 128
    xc_cap = rows_cap + 128

    mesh = plsc.VectorSubcoreMesh(core_axis_name="c", subcore_axis_name="s")

    def body(x_hbm, m_hbm, table_hbm, out_hbm,
             xv0, mv0, xc0, rows0, xv1, mv1, xc1, rows1, z0v, outb,
             gsem0, gsem1):
        wid = lax.axis_index("s") * 2 + lax.axis_index("c")
        bufs = ((xv0, mv0, xc0, rows0, gsem0), (xv1, mv1, xc1, rows1, gsem1))

        pltpu.sync_copy(table_hbm.at[pl.ds(0, 8)], z0v)
        lanes_i = lax.iota(jnp.int32, LANES)
        zeros_i = jnp.zeros((LANES,), jnp.int32)
        slack_fill = wid * LANES + lanes_i

        def load_and_compact(c, buf):
            xv, mv, xc, _, _ = buf
            base = (wid * rows_per_w + c * CHUNK_ROWS) * S
            pltpu.sync_copy(x_hbm.at[pl.ds(base, CS)], xv)
            pltpu.sync_copy(m_hbm.at[pl.ds(base, CS)], mv.at[pl.ds(0, CS)])
            ends = []
            cnts = []
            cur = jnp.int32(0)
            for r in range(CHUNK_ROWS):
                rb = r * S
                seg_start = cur
                for g in range(n_full_groups + 1):
                    if g < n_full_groups:
                        o = rb + g * LANES
                        mbool = mv[pl.ds(o, LANES)] > 0
                    else:
                        o = rb + S - LANES
                        mbool = jnp.logical_and(
                            mv[pl.ds(o, LANES)] > 0,
                            lanes_i >= (LANES - tail),
                        )
                    x16 = xv[pl.ds(o, LANES)]
                    plsc.store_compressed(
                        xc.at[pl.ds(cur, LANES)], x16, mask=mbool
                    )
                    pc = plsc.all_reduce_population_count(mbool)
                    cur = cur + pc[0]
                seg_len = cur - seg_start
                cnts.append(seg_len)
                xc[pl.ds(cur, LANES)] = zeros_i
                cur = seg_start + (
                    (seg_len + jnp.int32(UNROLL - 1)) & jnp.int32(-UNROLL)
                )
                ends.append(cur)
            for k in range(128 // LANES):
                xc[pl.ds(cur + k * LANES, LANES)] = slack_fill
            return tuple(ends) + tuple(cnts)

        def gather_copies(buf, total):
            xc, rows_v, gsem = buf[2], buf[3], buf[4]
            for s in range(n_slices):
                yield (
                    s * 128 < total,
                    pltpu.make_async_copy(
                        table_hbm.at[xc.at[pl.ds(s * 128, 128)]],
                        rows_v.at[pl.ds(s * 128, 128)],
                        gsem,
                    ),
                )

        def fire(buf, st):
            for pred, cp in gather_copies(buf, st[CHUNK_ROWS - 1]):
                @pl.when(pred)
                def _():
                    cp.start()

        def wait(buf, st):
            for pred, cp in gather_copies(buf, st[CHUNK_ROWS - 1]):
                @pl.when(pred)
                def _():
                    cp.wait()

        def process(c, buf, st):
            rows_v = buf[3]
            row0 = wid * rows_per_w + c * CHUNK_ROWS
            z0 = [z0v[0, pl.ds(d * LANES, LANES)] for d in range(d_regs)]
            z = jnp.zeros((LANES,), jnp.float32)
            for r in range(CHUNK_ROWS):
                lo = jnp.int32(0) if r == 0 else st[r - 1]
                hi = st[r]
                cnt = st[CHUNK_ROWS + r]

                def blk(b, accs):
                    base = lo + b * UNROLL
                    out = list(accs)
                    for j in range(UNROLL):
                        for d in range(d_regs):
                            out[d] = out[d] + rows_v[base + j, pl.ds(d * LANES, LANES)]
                    return tuple(out)

                nb = lax.shift_right_logical(hi - lo, 3)
                accs = lax.fori_loop(0, nb, blk, (z,) * d_regs)
                cntf = jnp.full(
                    (LANES,), cnt.astype(jnp.float32), jnp.float32
                )
                padf = jnp.full(
                    (LANES,), (hi - lo - cnt).astype(jnp.float32), jnp.float32
                )
                inv = jnp.float32(1.0) / (cntf + jnp.float32(1e-9))
                for d in range(d_regs):
                    outb[r, pl.ds(d * LANES, LANES)] = (
                        accs[d] - padf * z0[d]
                    ) * inv
            pltpu.sync_copy(outb, out_hbm.at[pl.ds(row0, CHUNK_ROWS)])

        st0 = load_and_compact(0, bufs[0])
        fire(bufs[0], st0)

        def pair_body(i, stA):
            c0 = 2 * i
            stB = load_and_compact(c0 + 1, bufs[1])
            fire(bufs[1], stB)
            wait(bufs[0], stA)
            process(c0, bufs[0], stA)
            c_next = jnp.minimum(jnp.int32(c0 + 2), jnp.int32(n_chunks - 1))
            stA2 = load_and_compact(c_next, bufs[0])
            fire(bufs[0], stA2)
            wait(bufs[1], stB)
            process(c0 + 1, bufs[1], stB)
            return stA2

        stF = lax.fori_loop(0, n_chunks // 2, pair_body, st0)
        wait(bufs[0], stF)

    return pl.kernel(
        body,
        out_type=jax.ShapeDtypeStruct((B, D), jnp.float32),
        mesh=mesh,
        compiler_params=pltpu.CompilerParams(
            use_tc_tiling_on_sc=False, needs_layout_passes=False
        ),
        scratch_types=[
            pltpu.VMEM((CS,), jnp.int32),
            pltpu.VMEM((CS + LANES,), jnp.float32),
            pltpu.VMEM((xc_cap,), jnp.int32),
            pltpu.VMEM((rows_cap, D), jnp.float32),
            pltpu.VMEM((CS,), jnp.int32),
            pltpu.VMEM((CS + LANES,), jnp.float32),
            pltpu.VMEM((xc_cap,), jnp.int32),
            pltpu.VMEM((rows_cap, D), jnp.float32),
            pltpu.VMEM((8, D), jnp.float32),
            pltpu.VMEM((CHUNK_ROWS, D), jnp.float32),
            pltpu.SemaphoreType.DMA,
            pltpu.SemaphoreType.DMA,
        ],
    )


@jax.jit
def kernel(x, mask, table):
    B, S = x.shape
    n_table, D = table.shape
    xf = x.reshape(-1).astype(jnp.int32)
    mf = mask.reshape(-1).astype(jnp.float32)
    return _build(B, S, D, n_table)(xf, mf, table)
